# baseline (speedup 1.0000x reference)
import jax
import jax.numpy as jnp
from jax.experimental import pallas as pl

_B = 4
_D, _H, _W = 11, 129, 129
_CIN, _COUT = 16, 32
_FL = 72
_NQ = 65 * _FL
_QA = 4753
_TAP = ((0, 0), (1, 0), (0, 1))


def _conv1_body(p_ref, w1_ref, y1_ref):
    d = pl.program_id(1)
    parts = []
    for kd in range(3):
        pz, sz = _TAP[kd]
        for kh in range(3):
            ph, sh = _TAP[kh]
            for kw in range(3):
                pw, sw = _TAP[kw]
                off = _FL * sh + sw
                parts.append(p_ref[0, pz, ph, pw, d + sz, :, off:off + _NQ])
    xt = jnp.concatenate(parts, axis=0)
    y = jax.lax.dot_general(xt, w1_ref[...], (((0,), (0,)), ((), ())),
                            preferred_element_type=jnp.float32)
    q = jax.lax.broadcasted_iota(jnp.int32, (_NQ, _COUT), 0)
    ok = (q % _FL < 64) & (q < 64 * _FL)
    y1_ref[0, 0] = jnp.where(ok, y, 0.0)


def _classes(mall):
    p = [[mall[:, 16 * (3 * kh + kw):16 * (3 * kh + kw) + 16]
          for kw in range(3)] for kh in range(3)]

    def sh(x, k):
        z = jnp.zeros((k, _CIN), jnp.float32)
        return jnp.concatenate([z, x[:_NQ - k]], axis=0)

    cee = p[0][0] + sh(p[0][2], 1) + sh(p[2][0], _FL) + sh(p[2][2], _FL + 1)
    ceo = p[0][1] + sh(p[2][1], _FL)
    coe = p[1][0] + sh(p[1][2], 1)
    coo = p[1][1]
    return cee, ceo, coe, coo


def _deconv_body(ya_ref, yb_ref, w02_ref, w1o_ref, o_ref):
    dz = pl.program_id(1)

    def emit(mall):
        cee, ceo, coe, coo = _classes(mall)
        o_ref[0, 0, 0, 0] = cee
        o_ref[0, 0, 0, 1] = ceo
        o_ref[0, 0, 1, 0] = coe
        o_ref[0, 0, 1, 1] = coo

    @pl.when(dz % 2 == 0)
    def _even():
        va = jnp.where(dz < 10, 1.0, 0.0).astype(jnp.float32)
        vb = jnp.where(dz >= 2, 1.0, 0.0).astype(jnp.float32)
        ya = ya_ref[0, 0] * va
        yb = yb_ref[0, 0] * vb
        emit(jnp.dot(jnp.concatenate([ya, yb], axis=1), w02_ref[...],
                     preferred_element_type=jnp.float32))

    @pl.when(dz % 2 == 1)
    def _odd():
        emit(jnp.dot(ya_ref[0, 0], w1o_ref[...],
                     preferred_element_type=jnp.float32))


def kernel(features, coors, batch_size, W1, W2):
    coors = coors.astype(jnp.int32)
    bi, zi, yi, xi = coors[:, 0], coors[:, 1], coors[:, 2], coors[:, 3]
    n = features.shape[0]
    valid = (bi < batch_size).astype(features.dtype)
    f = features * valid[:, None]

    idx1 = jnp.arange(1, n + 1, dtype=jnp.int32)
    win = jnp.zeros((_B, _D, _H, _W), jnp.int32).at[bi, zi, yi, xi].max(idx1)
    owner = (win[bi, zi, yi, xi] == idx1).astype(features.dtype)
    fd = f * owner[:, None]

    qi = (yi // 2) * _FL + xi // 2
    P = jnp.zeros((_B, 2, 2, 2, 6, _CIN, _QA), jnp.float32).at[
        bi, zi % 2, yi % 2, xi % 2, zi // 2, :, qi].add(fd)
    maskf = (win > 0).astype(jnp.float32)

    W1r = W1.reshape(27 * _CIN, _COUT)
    W2f = W2[::-1, ::-1, ::-1, :, :]
    w2cat = [W2f[kz].transpose(2, 0, 1, 3).reshape(_COUT, 9 * _CIN)
             for kz in range(3)]
    W2cat02 = jnp.concatenate([w2cat[0], w2cat[2]], axis=0)
    W2cat1 = w2cat[1]

    y1 = pl.pallas_call(
        _conv1_body,
        grid=(_B, 5),
        in_specs=[
            pl.BlockSpec((1, 2, 2, 2, 6, _CIN, _QA),
                         lambda b, d: (b, 0, 0, 0, 0, 0, 0)),
            pl.BlockSpec((27 * _CIN, _COUT), lambda b, d: (0, 0)),
        ],
        out_specs=pl.BlockSpec((1, 1, _NQ, _COUT),
                               lambda b, d: (b, d, 0, 0)),
        out_shape=jax.ShapeDtypeStruct((_B, 5, _NQ, _COUT), jnp.float32),
    )(P, W1r)

    o_par = pl.pallas_call(
        _deconv_body,
        grid=(_B, _D),
        in_specs=[
            pl.BlockSpec((1, 1, _NQ, _COUT),
                         lambda b, z: (b, jnp.clip(z // 2, 0, 4), 0, 0)),
            pl.BlockSpec((1, 1, _NQ, _COUT),
                         lambda b, z: (b, jnp.clip(z // 2 - 1, 0, 4), 0, 0)),
            pl.BlockSpec((2 * _COUT, 9 * _CIN), lambda b, z: (0, 0)),
            pl.BlockSpec((_COUT, 9 * _CIN), lambda b, z: (0, 0)),
        ],
        out_specs=pl.BlockSpec((1, 1, 2, 2, _NQ, _CIN),
                               lambda b, z: (b, z, 0, 0, 0, 0)),
        out_shape=jax.ShapeDtypeStruct((_B, _D, 2, 2, _NQ, _CIN),
                                       jnp.float32),
    )(y1, y1, W2cat02, W2cat1)

    op7 = o_par.reshape(_B, _D, 2, 2, 65, _FL, _CIN)[..., :65, :]
    t = jnp.transpose(op7, (0, 6, 1, 4, 2, 5, 3)).reshape(
        _B, _CIN, _D, 130, 130)[:, :, :, :129, :129]
    return t * maskf[:, None]
